# baseline (speedup 1.0000x reference)
import functools

import jax
import jax.numpy as jnp
from jax import lax
from jax.experimental import pallas as pl
from jax.experimental.pallas import tpu as pltpu
from jax.experimental.pallas import tpu_sc as plsc

N = 10000
E = 320000
D_FEAT = 128
D_EDGE = 16

NC = 2
NS = 16
NW = NC * NS
CHUNK = 128
NCHUNKS = E // CHUNK
CPT = NCHUNKS // NW
XTRA = NCHUNKS - CPT * NW
BPB = 6
NBLK = CPT // BPB
BLK = BPB * CHUNK
N_PAD = 10112
ROWS_PER_SUB = N_PAD // NS

_sc_mesh = plsc.VectorSubcoreMesh(core_axis_name="c", subcore_axis_name="s")


@functools.partial(
    pl.kernel,
    out_type=jax.ShapeDtypeStruct((NC, N_PAD, D_EDGE), jnp.float32),
    mesh=_sc_mesh,
    compiler_params=pltpu.CompilerParams(use_tc_tiling_on_sc=False),
    scratch_types=[
        pltpu.VMEM_SHARED((N_PAD, D_EDGE), jnp.float32),
        pltpu.VMEM((3, BLK, D_EDGE), jnp.float32),
        pltpu.VMEM((3, BLK), jnp.int32),
        pltpu.SemaphoreType.DMA,
        pltpu.SemaphoreType.DMA,
        pltpu.SemaphoreType.DMA,
        pltpu.SemaphoreType.DMA,
        pltpu.SemaphoreType.DMA,
        pltpu.SemaphoreType.DMA,
    ],
)
def _sc_segment_sum(z_hbm, ea_hbm, recv_hbm, out_hbm, acc, rows3, idx3,
                    lsem0, lsem1, lsem2, ssem0, ssem1, ssem2):
    c = lax.axis_index("c")
    s = lax.axis_index("s")
    wid = s * NC + c
    base = wid * CPT * CHUNK
    row0 = s * ROWS_PER_SUB

    lsems = (lsem0, lsem1, lsem2)
    ssems = (ssem0, ssem1, ssem2)
    loads = [None] * NBLK
    scats = [[] for _ in range(NBLK)]

    def start_load(blk):
        buf = blk % 3
        off = base + blk * BLK
        di = pltpu.async_copy(recv_hbm.at[pl.ds(off, BLK)], idx3.at[buf],
                              lsems[buf])
        dr = pltpu.async_copy(ea_hbm.at[pl.ds(off, BLK)], rows3.at[buf],
                              lsems[buf])
        loads[blk] = (di, dr)

    def fire_scatters(blk):
        buf = blk % 3
        for j in range(BPB):
            s0 = j * CHUNK
            d = pltpu.async_copy(
                rows3.at[buf].at[pl.ds(s0, CHUNK)],
                acc.at[idx3.at[buf].at[pl.ds(s0, CHUNK)]],
                ssems[buf], add=True)
            scats[blk].append(d)

    start_load(0)
    start_load(1)
    start_load(2)
    pltpu.sync_copy(z_hbm.at[pl.ds(row0, ROWS_PER_SUB)],
                    acc.at[pl.ds(row0, ROWS_PER_SUB)])
    plsc.subcore_barrier()

    for blk in range(NBLK):
        di, dr = loads[blk]
        di.wait()
        dr.wait()
        fire_scatters(blk)
        if blk >= 1 and blk + 2 < NBLK:
            for d in scats[blk - 1]:
                d.wait()
            start_load(blk + 2)
    for blk in range(NBLK - 3, NBLK):
        for d in scats[blk]:
            d.wait()

    def _extra(_, __):
        off = (NW * CPT + wid) * CHUNK
        pltpu.sync_copy(recv_hbm.at[pl.ds(off, CHUNK)],
                        idx3.at[0].at[pl.ds(0, CHUNK)])
        pltpu.sync_copy(ea_hbm.at[pl.ds(off, CHUNK)],
                        rows3.at[0].at[pl.ds(0, CHUNK)])
        pltpu.sync_copy(rows3.at[0].at[pl.ds(0, CHUNK)],
                        acc.at[idx3.at[0].at[pl.ds(0, CHUNK)]], add=True)
        return 0

    lax.fori_loop(0, jnp.where(wid < XTRA, 1, 0), _extra, 0)

    plsc.subcore_barrier()

    pltpu.sync_copy(acc.at[pl.ds(row0, ROWS_PER_SUB)],
                    out_hbm.at[c, pl.ds(row0, ROWS_PER_SUB)])


def _mlp_body(x_ref, p0_ref, p1_ref, wx_ref, we_ref, b_ref, o_ref):
    agg = p0_ref[...] + p1_ref[...]
    o_ref[...] = (
        jnp.dot(x_ref[...], wx_ref[...], preferred_element_type=jnp.float32)
        + jnp.dot(agg, we_ref[...], preferred_element_type=jnp.float32)
        + b_ref[...]
    )


@jax.jit
def _tc_mlp(x, p0, p1, wx, we, b2d):
    return pl.pallas_call(
        _mlp_body,
        out_shape=jax.ShapeDtypeStruct((N, D_FEAT), jnp.float32),
    )(x, p0, p1, wx, we, b2d)


@jax.jit
def kernel(x, edge_index, edge_attr, pos, W, b):
    recv = edge_index[1]
    zrows = jnp.zeros((N_PAD, D_EDGE), jnp.float32)
    partials = _sc_segment_sum(zrows, edge_attr, recv)
    p0 = partials[0, :N]
    p1 = partials[1, :N]
    x_ = _tc_mlp(x, p0, p1, W[:D_FEAT], W[D_FEAT:], b[None, :])
    return (x_, edge_attr, edge_index, pos)


# speedup vs baseline: 1.0030x; 1.0030x over previous
import functools

import jax
import jax.numpy as jnp
from jax import lax
from jax.experimental import pallas as pl
from jax.experimental.pallas import tpu as pltpu
from jax.experimental.pallas import tpu_sc as plsc

N = 10000
E = 320000
D_FEAT = 128
D_EDGE = 16

NC = 2
NS = 16
NW = NC * NS
CHUNK = 128
NCHUNKS = E // CHUNK
CPT = NCHUNKS // NW
XTRA = NCHUNKS - CPT * NW
BPB = 6
NBLK = CPT // BPB
BLK = BPB * CHUNK
N_PAD = 10112
ROWS_PER_SUB = N_PAD // NS

_sc_mesh = plsc.VectorSubcoreMesh(core_axis_name="c", subcore_axis_name="s")


@functools.partial(
    pl.kernel,
    out_type=jax.ShapeDtypeStruct((NC, N_PAD, D_EDGE), jnp.float32),
    mesh=_sc_mesh,
    compiler_params=pltpu.CompilerParams(use_tc_tiling_on_sc=False),
    scratch_types=[
        pltpu.VMEM_SHARED((N_PAD, D_EDGE), jnp.float32),
        pltpu.VMEM((BLK, D_EDGE), jnp.float32),
        pltpu.VMEM((BLK, D_EDGE), jnp.float32),
        pltpu.VMEM((BLK,), jnp.int32),
        pltpu.VMEM((BLK,), jnp.int32),
        pltpu.SemaphoreType.DMA,
        pltpu.SemaphoreType.DMA,
    ],
)
def _sc_segment_sum(z_hbm, ea_hbm, recv_hbm, out_hbm, acc, rows_a, rows_b,
                    idx_a, idx_b, sem_a, sem_b):
    c = lax.axis_index("c")
    s = lax.axis_index("s")
    wid = s * NC + c
    base = wid * CPT * CHUNK
    row0 = s * ROWS_PER_SUB

    row_bufs = (rows_a, rows_b)
    idx_bufs = (idx_a, idx_b)
    sems = (sem_a, sem_b)

    def start_load(blk, buf):
        off = base + blk * BLK
        pltpu.async_copy(recv_hbm.at[pl.ds(off, BLK)], idx_bufs[buf],
                         sems[buf])
        pltpu.async_copy(ea_hbm.at[pl.ds(off, BLK)], row_bufs[buf],
                         sems[buf])

    def wait_load(buf):
        pltpu.make_async_copy(recv_hbm.at[pl.ds(0, BLK)], idx_bufs[buf],
                              sems[buf]).wait()
        pltpu.make_async_copy(ea_hbm.at[pl.ds(0, BLK)], row_bufs[buf],
                              sems[buf]).wait()

    def scatter_block(buf):
        for j in range(BPB):
            s0 = j * CHUNK
            pltpu.sync_copy(row_bufs[buf].at[pl.ds(s0, CHUNK)],
                            acc.at[idx_bufs[buf].at[pl.ds(s0, CHUNK)]],
                            add=True)

    start_load(0, 0)
    pltpu.sync_copy(z_hbm.at[pl.ds(row0, ROWS_PER_SUB)],
                    acc.at[pl.ds(row0, ROWS_PER_SUB)])
    plsc.subcore_barrier()

    def _pair(i, _):
        g = 2 * i
        start_load(g + 1, 1)
        wait_load(0)
        scatter_block(0)
        start_load(jnp.minimum(g + 2, NBLK - 1), 0)
        wait_load(1)
        scatter_block(1)
        return 0

    lax.fori_loop(0, NBLK // 2, _pair, 0)
    wait_load(0)
    scatter_block(0)

    def _extra(_, __):
        off = (NW * CPT + wid) * CHUNK
        pltpu.sync_copy(recv_hbm.at[pl.ds(off, CHUNK)],
                        idx_a.at[pl.ds(0, CHUNK)])
        pltpu.sync_copy(ea_hbm.at[pl.ds(off, CHUNK)],
                        rows_a.at[pl.ds(0, CHUNK)])
        pltpu.sync_copy(rows_a.at[pl.ds(0, CHUNK)],
                        acc.at[idx_a.at[pl.ds(0, CHUNK)]], add=True)
        return 0

    lax.fori_loop(0, jnp.where(wid < XTRA, 1, 0), _extra, 0)

    plsc.subcore_barrier()

    pltpu.sync_copy(acc.at[pl.ds(row0, ROWS_PER_SUB)],
                    out_hbm.at[c, pl.ds(row0, ROWS_PER_SUB)])


def _mlp_body(x_ref, p0_ref, p1_ref, wx_ref, we_ref, b_ref, o_ref):
    agg = p0_ref[...] + p1_ref[...]
    o_ref[...] = (
        jnp.dot(x_ref[...], wx_ref[...], preferred_element_type=jnp.float32)
        + jnp.dot(agg, we_ref[...], preferred_element_type=jnp.float32)
        + b_ref[...]
    )


@jax.jit
def _tc_mlp(x, p0, p1, wx, we, b2d):
    return pl.pallas_call(
        _mlp_body,
        out_shape=jax.ShapeDtypeStruct((N, D_FEAT), jnp.float32),
    )(x, p0, p1, wx, we, b2d)


@jax.jit
def kernel(x, edge_index, edge_attr, pos, W, b):
    recv = edge_index[1]
    zrows = jnp.zeros((N_PAD, D_EDGE), jnp.float32)
    partials = _sc_segment_sum(zrows, edge_attr, recv)
    p0 = partials[0, :N]
    p1 = partials[1, :N]
    x_ = _tc_mlp(x, p0, p1, W[:D_FEAT], W[D_FEAT:], b[None, :])
    return (x_, edge_attr, edge_index, pos)
